# baseline (speedup 1.0000x reference)
import functools

import jax
import jax.numpy as jnp
from jax import lax
from jax.experimental import pallas as pl
from jax.experimental.pallas import tpu as pltpu
from jax.experimental.pallas import tpu_sc as plsc

D = 768
F = 768
E = 8
S = 2048
T = 64
SPAD = S + E * T
NB = SPAD // T
NC = 2
NS = 16
NW = NC * NS
RPW = SPAD // NW
CHUNK = 512


_M16 = -65536


def _router_kernel(x_ref, wr_ref, src_ref, dst_ref, bexp_ref, xp_ref):
    x = x_ref[...]
    xb = lax.bitcast_convert_type(
        x.astype(jnp.bfloat16).astype(jnp.float32), jnp.int32)
    xp_ref[...] = ((xb[:, D // 2:] & _M16)
                   | lax.shift_right_logical(xb[:, :D // 2], 16))
    logits = jnp.dot(x, wr_ref[...],
                     preferred_element_type=jnp.float32)
    m = jnp.max(logits, axis=1, keepdims=True)
    ex = jnp.exp(logits - m)
    probs = ex / jnp.sum(ex, axis=1, keepdims=True)
    pm = jnp.max(probs, axis=1, keepdims=True)
    eidx = lax.broadcasted_iota(jnp.int32, (S, E), 1)
    ids2 = jnp.where(probs == pm, eidx, E)
    eid = jnp.min(ids2, axis=1)
    oh = (eid[:, None] == eidx).astype(jnp.float32)

    counts = jnp.round(jnp.sum(oh, axis=0)).astype(jnp.int32)
    padded = ((counts + T - 1) // T) * T
    lt = (lax.broadcasted_iota(jnp.int32, (E, E), 1)
          < lax.broadcasted_iota(jnp.int32, (E, E), 0))
    starts = jnp.sum(jnp.where(lt, padded[None, :], 0), axis=1)
    cum_incl = starts + padded

    r128 = lax.broadcasted_iota(jnp.int32, (T, T), 0)
    c128 = lax.broadcasted_iota(jnp.int32, (T, T), 1)
    tri = (c128 < r128).astype(jnp.bfloat16)
    base = jnp.zeros((1, E), jnp.float32)
    pos_parts = []
    startsf = starts.astype(jnp.float32)
    for k in range(S // T):
        ohk = oh[k * T:(k + 1) * T]
        rk = jnp.dot(tri, ohk.astype(jnp.bfloat16),
                     preferred_element_type=jnp.float32)
        full = rk + base + startsf[None, :]
        pos_parts.append(jnp.sum(full * ohk, axis=1))
        base = base + jnp.sum(ohk, axis=0, keepdims=True)
    pos = jnp.concatenate(pos_parts, axis=0)

    ti = lax.broadcasted_iota(jnp.int32, (1, S), 1)
    tq = (ti // 16).astype(jnp.float32)
    tr = (ti % 16).astype(jnp.float32)
    ones = jnp.ones((1, S), jnp.float32)
    lhs = jnp.concatenate([tq, tr, ones], axis=0).astype(jnp.bfloat16)
    src_parts, dst_parts = [], []
    for c in range(SPAD // CHUNK):
        cols = (c * CHUNK
                + lax.broadcasted_iota(jnp.int32, (1, CHUNK), 1)
                ).astype(jnp.float32)
        mm = (pos[:, None] == cols).astype(jnp.bfloat16)
        acc = jnp.dot(lhs, mm, preferred_element_type=jnp.float32)
        srcc = 16.0 * acc[0:1] + acc[1:2]
        dstc = srcc + (1.0 - acc[2:3]) * (float(S) + jnp.mod(cols, 8.0))
        src_parts.append(srcc)
        dst_parts.append(dstc)
    src = jnp.concatenate(src_parts, axis=1)
    dst = jnp.concatenate(dst_parts, axis=1)
    src_ref[...] = jnp.round(src).astype(jnp.int32)
    dst_ref[...] = jnp.round(dst).astype(jnp.int32)

    gt = lax.broadcasted_iota(jnp.int32, (1, NB), 1) * T
    ge = (gt[:, :, None] >= cum_incl[None, None, :]).astype(jnp.int32)
    bexp_ref[0:1, :] = jnp.minimum(jnp.sum(ge, axis=2), E - 1)
    bexp_ref[1:2, :] = (gt < cum_incl[E - 1]).astype(jnp.int32)


def _router_call(x2, wr):
    return pl.pallas_call(
        _router_kernel,
        out_shape=(
            jax.ShapeDtypeStruct((1, SPAD), jnp.int32),
            jax.ShapeDtypeStruct((1, SPAD), jnp.int32),
            jax.ShapeDtypeStruct((2, NB), jnp.int32),
            jax.ShapeDtypeStruct((S, D // 2), jnp.int32),
        ),
    )(x2, wr)


D2 = D // 2
NCH = 5
CH = RPW // NCH


@functools.cache
def _sc_kernels():
    mesh = plsc.VectorSubcoreMesh(core_axis_name="c", subcore_axis_name="s")

    @functools.partial(
        pl.kernel,
        mesh=mesh,
        out_type=jax.ShapeDtypeStruct((SPAD, D2), jnp.int32),
        scratch_types=[
            pltpu.VMEM((NCH, CH), jnp.int32),
            pltpu.VMEM((NCH, CH, D2), jnp.int32),
            pltpu.SemaphoreType.DMA,
            pltpu.SemaphoreType.DMA,
        ],
    )
    def _sc_gather(src_hbm, x_hbm, out_hbm, idx_v, rows_v, gsem, osem):
        wid = lax.axis_index("s") * NC + lax.axis_index("c")
        base = wid * RPW
        pltpu.sync_copy(src_hbm.at[wid], idx_v)
        gathers = []
        for c in range(NCH):
            gathers.append(
                pltpu.async_copy(x_hbm.at[idx_v.at[c]], rows_v.at[c], gsem))
        writes = []
        for c in range(NCH):
            gathers[c].wait()
            writes.append(pltpu.async_copy(
                rows_v.at[c], out_hbm.at[pl.ds(base + c * CH, CH)], osem))
        for w in writes:
            w.wait()

    @functools.partial(
        pl.kernel,
        mesh=mesh,
        out_type=jax.ShapeDtypeStruct((S + 8, D2), jnp.int32),
        scratch_types=[
            pltpu.VMEM((NCH, CH), jnp.int32),
            pltpu.VMEM((NCH, CH, D2), jnp.int32),
            pltpu.SemaphoreType.DMA,
            pltpu.SemaphoreType.DMA,
        ],
    )
    def _sc_scatter(ys_hbm, dst_hbm, out_hbm, idx_v, rows_v, rsem, wsem):
        wid = lax.axis_index("s") * NC + lax.axis_index("c")
        base = wid * RPW
        pltpu.sync_copy(dst_hbm.at[wid], idx_v)
        reads = []
        for c in range(NCH):
            reads.append(pltpu.async_copy(
                ys_hbm.at[pl.ds(base + c * CH, CH)], rows_v.at[c], rsem))
        writes = []
        for c in range(NCH):
            reads[c].wait()
            writes.append(
                pltpu.async_copy(rows_v.at[c], out_hbm.at[idx_v.at[c]], wsem))
        for w in writes:
            w.wait()

    return _sc_gather, _sc_scatter


def _unpack_f32(pk):
    lo = lax.bitcast_convert_type(lax.shift_left(pk, 16), jnp.float32)
    hi = lax.bitcast_convert_type(pk & _M16, jnp.float32)
    return jnp.concatenate([lo, hi], axis=1)


def _pack_i32(y):
    yb = lax.bitcast_convert_type(
        y.astype(jnp.bfloat16).astype(jnp.float32), jnp.int32)
    return ((yb[:, D // 2:] & _M16)
            | lax.shift_right_logical(yb[:, :D // 2], 16))


def _gmm_kernel(bexp_ref, xs_ref, wg_ref, wu_ref, wd_ref, ys_ref):
    @pl.when(bexp_ref[1, pl.program_id(0)] == 1)
    def _():
        xb = _unpack_f32(xs_ref[...]).astype(jnp.bfloat16)
        g = jnp.dot(xb, wg_ref[0].astype(jnp.bfloat16),
                    preferred_element_type=jnp.float32)
        u = jnp.dot(xb, wu_ref[0].astype(jnp.bfloat16),
                    preferred_element_type=jnp.float32)
        h = (g * lax.logistic(g) * u).astype(jnp.bfloat16)
        y = jnp.dot(h, wd_ref[0].astype(jnp.bfloat16),
                    preferred_element_type=jnp.float32)
        ys_ref[...] = _pack_i32(y)


def _gmm_call(bexp, xs, wg, wu, wd):
    grid_spec = pltpu.PrefetchScalarGridSpec(
        num_scalar_prefetch=1,
        grid=(NB,),
        in_specs=[
            pl.BlockSpec((T, D // 2), lambda g, be: (g, 0)),
            pl.BlockSpec((1, D, F), lambda g, be: (be[0, g], 0, 0)),
            pl.BlockSpec((1, D, F), lambda g, be: (be[0, g], 0, 0)),
            pl.BlockSpec((1, F, D), lambda g, be: (be[0, g], 0, 0)),
        ],
        out_specs=pl.BlockSpec((T, D // 2), lambda g, be: (g, 0)),
    )
    return pl.pallas_call(
        _gmm_kernel,
        grid_spec=grid_spec,
        out_shape=jax.ShapeDtypeStruct((SPAD, D // 2), jnp.int32),
    )(bexp, xs, wg, wu, wd)


_TB = 256

def _shared_kernel(x_ref, wsg_ref, wsu_ref, wsd_ref, y_ref):
    xb = x_ref[...].astype(jnp.bfloat16)
    g = jnp.dot(xb, wsg_ref[...].astype(jnp.bfloat16),
                preferred_element_type=jnp.float32)
    u = jnp.dot(xb, wsu_ref[...].astype(jnp.bfloat16),
                preferred_element_type=jnp.float32)
    h = (g * lax.logistic(g) * u).astype(jnp.bfloat16)
    y_ref[...] = jnp.dot(h, wsd_ref[...].astype(jnp.bfloat16),
                         preferred_element_type=jnp.float32)


def _shared_call(x2, wsg, wsu, wsd):
    return pl.pallas_call(
        _shared_kernel,
        grid=(S // _TB,),
        in_specs=[
            pl.BlockSpec((_TB, D), lambda g: (g, 0)),
            pl.BlockSpec((D, F), lambda g: (0, 0)),
            pl.BlockSpec((D, F), lambda g: (0, 0)),
            pl.BlockSpec((F, D), lambda g: (0, 0)),
        ],
        out_specs=pl.BlockSpec((_TB, D), lambda g: (g, 0)),
        out_shape=jax.ShapeDtypeStruct((S, D), jnp.float32),
    )(x2, wsg, wsu, wsd)


def _add_kernel(ysh_ref, yb_ref, y_ref):
    y_ref[...] = ysh_ref[...] + _unpack_f32(yb_ref[...])


def _add_call(ysh, yb):
    return pl.pallas_call(
        _add_kernel,
        grid=(S // _TB,),
        in_specs=[
            pl.BlockSpec((_TB, D), lambda g: (g, 0)),
            pl.BlockSpec((_TB, D // 2), lambda g: (g, 0)),
        ],
        out_specs=pl.BlockSpec((_TB, D), lambda g: (g, 0)),
        out_shape=jax.ShapeDtypeStruct((S, D), jnp.float32),
    )(ysh, yb)


def kernel(x, Wr, Wg, Wu, Wd, Wsg, Wsu, Wsd):
    x2 = x.reshape(S, D)
    _sc_gather, _sc_scatter = _sc_kernels()
    src, dst, bexp, xp = _router_call(x2, Wr)
    ysh = _shared_call(x2, Wsg, Wsu, Wsd)
    xs = _sc_gather(src.reshape(NW, NCH, CH), xp)
    ys = _gmm_call(bexp, xs, Wg, Wu, Wd)
    ybuf = _sc_scatter(ys, dst.reshape(NW, NCH, CH))
    y = _add_call(ysh, ybuf[:S])
    return y.reshape(1, S, D)


# speedup vs baseline: 1.2091x; 1.2091x over previous
import functools

import jax
import jax.numpy as jnp
from jax import lax
from jax.experimental import pallas as pl
from jax.experimental.pallas import tpu as pltpu
from jax.experimental.pallas import tpu_sc as plsc

D = 768
F = 768
E = 8
S = 2048
T = 64
SPAD = S + E * T
NB = SPAD // T
NC = 2
NS = 16
NW = NC * NS
RPW = SPAD // NW
CHUNK = 512


_M16 = -65536


def _router_kernel(x_ref, wr_ref, pos_ref, dst_ref, bexp_ref, xp_ref):
    x = x_ref[...]
    xb = lax.bitcast_convert_type(
        x.astype(jnp.bfloat16).astype(jnp.float32), jnp.int32)
    xp_ref[...] = ((xb[:, D // 2:] & _M16)
                   | lax.shift_right_logical(xb[:, :D // 2], 16))
    logits = jnp.dot(x, wr_ref[...],
                     preferred_element_type=jnp.float32)
    m = jnp.max(logits, axis=1, keepdims=True)
    ex = jnp.exp(logits - m)
    probs = ex / jnp.sum(ex, axis=1, keepdims=True)
    pm = jnp.max(probs, axis=1, keepdims=True)
    eidx = lax.broadcasted_iota(jnp.int32, (S, E), 1)
    ids2 = jnp.where(probs == pm, eidx, E)
    eid = jnp.min(ids2, axis=1)
    oh = (eid[:, None] == eidx).astype(jnp.float32)

    counts = jnp.round(jnp.sum(oh, axis=0)).astype(jnp.int32)
    padded = ((counts + T - 1) // T) * T
    lt = (lax.broadcasted_iota(jnp.int32, (E, E), 1)
          < lax.broadcasted_iota(jnp.int32, (E, E), 0))
    starts = jnp.sum(jnp.where(lt, padded[None, :], 0), axis=1)
    cum_incl = starts + padded

    r128 = lax.broadcasted_iota(jnp.int32, (T, T), 0)
    c128 = lax.broadcasted_iota(jnp.int32, (T, T), 1)
    tri = (c128 < r128).astype(jnp.bfloat16)
    base = jnp.zeros((1, E), jnp.float32)
    pos_parts = []
    startsf = starts.astype(jnp.float32)
    for k in range(S // T):
        ohk = oh[k * T:(k + 1) * T]
        rk = jnp.dot(tri, ohk.astype(jnp.bfloat16),
                     preferred_element_type=jnp.float32)
        full = rk + base + startsf[None, :]
        pos_parts.append(jnp.sum(full * ohk, axis=1))
        base = base + jnp.sum(ohk, axis=0, keepdims=True)
    pos = jnp.concatenate(pos_parts, axis=0)
    pos_ref[...] = jnp.round(pos)[None, :].astype(jnp.int32)

    ti = lax.broadcasted_iota(jnp.int32, (1, S), 1)
    tq = (ti // 16).astype(jnp.float32)
    tr = (ti % 16).astype(jnp.float32)
    ones = jnp.ones((1, S), jnp.float32)
    lhs = jnp.concatenate([tq, tr, ones], axis=0).astype(jnp.bfloat16)
    dst_parts = []
    for c in range(SPAD // CHUNK):
        cols = (c * CHUNK
                + lax.broadcasted_iota(jnp.int32, (1, CHUNK), 1)
                ).astype(jnp.float32)
        mm = (pos[:, None] == cols).astype(jnp.bfloat16)
        acc = jnp.dot(lhs, mm, preferred_element_type=jnp.float32)
        srcc = 16.0 * acc[0:1] + acc[1:2]
        dstc = srcc + (1.0 - acc[2:3]) * (float(S) + jnp.mod(cols, 8.0))
        dst_parts.append(dstc)
    dst = jnp.concatenate(dst_parts, axis=1)
    dst_ref[...] = jnp.round(dst).astype(jnp.int32)

    gt = lax.broadcasted_iota(jnp.int32, (1, NB), 1) * T
    ge = (gt[:, :, None] >= cum_incl[None, None, :]).astype(jnp.int32)
    bexp_ref[0:1, :] = jnp.minimum(jnp.sum(ge, axis=2), E - 1)
    bexp_ref[1:2, :] = (gt < cum_incl[E - 1]).astype(jnp.int32)


def _router_call(x2, wr):
    return pl.pallas_call(
        _router_kernel,
        out_shape=(
            jax.ShapeDtypeStruct((1, S), jnp.int32),
            jax.ShapeDtypeStruct((1, SPAD), jnp.int32),
            jax.ShapeDtypeStruct((2, NB), jnp.int32),
            jax.ShapeDtypeStruct((S, D // 2), jnp.int32),
        ),
    )(x2, wr)


D2 = D // 2
NCH = 5
CH = RPW // NCH
TPW = S // NW
NCG = 4
CG = TPW // NCG


@functools.cache
def _sc_kernels():
    mesh = plsc.VectorSubcoreMesh(core_axis_name="c", subcore_axis_name="s")

    @functools.partial(
        pl.kernel,
        mesh=mesh,
        out_type=jax.ShapeDtypeStruct((SPAD, D2), jnp.int32),
        scratch_types=[
            pltpu.VMEM((NCG, CG), jnp.int32),
            pltpu.VMEM((NCG, CG, D2), jnp.int32),
            pltpu.SemaphoreType.DMA,
            pltpu.SemaphoreType.DMA,
        ],
    )
    def _sc_gather(pos_hbm, x_hbm, out_hbm, idx_v, rows_v, rsem, wsem):
        wid = lax.axis_index("s") * NC + lax.axis_index("c")
        base = wid * TPW
        pltpu.sync_copy(pos_hbm.at[wid], idx_v)
        reads = []
        for c in range(NCG):
            reads.append(pltpu.async_copy(
                x_hbm.at[pl.ds(base + c * CG, CG)], rows_v.at[c], rsem))
        writes = []
        for c in range(NCG):
            reads[c].wait()
            writes.append(
                pltpu.async_copy(rows_v.at[c], out_hbm.at[idx_v.at[c]], wsem))
        for w in writes:
            w.wait()

    @functools.partial(
        pl.kernel,
        mesh=mesh,
        out_type=jax.ShapeDtypeStruct((S + 8, D2), jnp.int32),
        scratch_types=[
            pltpu.VMEM((NCH, CH), jnp.int32),
            pltpu.VMEM((NCH, CH, D2), jnp.int32),
            pltpu.SemaphoreType.DMA,
            pltpu.SemaphoreType.DMA,
        ],
    )
    def _sc_scatter(ys_hbm, dst_hbm, out_hbm, idx_v, rows_v, rsem, wsem):
        wid = lax.axis_index("s") * NC + lax.axis_index("c")
        base = wid * RPW
        pltpu.sync_copy(dst_hbm.at[wid], idx_v)
        reads = []
        for c in range(NCH):
            reads.append(pltpu.async_copy(
                ys_hbm.at[pl.ds(base + c * CH, CH)], rows_v.at[c], rsem))
        writes = []
        for c in range(NCH):
            reads[c].wait()
            writes.append(
                pltpu.async_copy(rows_v.at[c], out_hbm.at[idx_v.at[c]], wsem))
        for w in writes:
            w.wait()

    return _sc_gather, _sc_scatter


def _unpack_f32(pk):
    lo = lax.bitcast_convert_type(lax.shift_left(pk, 16), jnp.float32)
    hi = lax.bitcast_convert_type(pk & _M16, jnp.float32)
    return jnp.concatenate([lo, hi], axis=1)


def _pack_i32(y):
    yb = lax.bitcast_convert_type(
        y.astype(jnp.bfloat16).astype(jnp.float32), jnp.int32)
    return ((yb[:, D // 2:] & _M16)
            | lax.shift_right_logical(yb[:, :D // 2], 16))


def _gmm_kernel(bexp_ref, xs_ref, wg_ref, wu_ref, wd_ref, ys_ref):
    @pl.when(bexp_ref[1, pl.program_id(0)] == 1)
    def _():
        xb = _unpack_f32(xs_ref[...]).astype(jnp.bfloat16)
        g = jnp.dot(xb, wg_ref[0].astype(jnp.bfloat16),
                    preferred_element_type=jnp.float32)
        u = jnp.dot(xb, wu_ref[0].astype(jnp.bfloat16),
                    preferred_element_type=jnp.float32)
        h = (g * lax.logistic(g) * u).astype(jnp.bfloat16)
        y = jnp.dot(h, wd_ref[0].astype(jnp.bfloat16),
                    preferred_element_type=jnp.float32)
        ys_ref[...] = _pack_i32(y)


def _gmm_call(bexp, xs, wg, wu, wd):
    grid_spec = pltpu.PrefetchScalarGridSpec(
        num_scalar_prefetch=1,
        grid=(NB,),
        in_specs=[
            pl.BlockSpec((T, D // 2), lambda g, be: (g, 0)),
            pl.BlockSpec((1, D, F), lambda g, be: (be[0, g], 0, 0)),
            pl.BlockSpec((1, D, F), lambda g, be: (be[0, g], 0, 0)),
            pl.BlockSpec((1, F, D), lambda g, be: (be[0, g], 0, 0)),
        ],
        out_specs=pl.BlockSpec((T, D // 2), lambda g, be: (g, 0)),
    )
    return pl.pallas_call(
        _gmm_kernel,
        grid_spec=grid_spec,
        out_shape=jax.ShapeDtypeStruct((SPAD, D // 2), jnp.int32),
    )(bexp, xs, wg, wu, wd)


_TB = 256

def _shared_kernel(x_ref, wsg_ref, wsu_ref, wsd_ref, y_ref):
    xb = x_ref[...].astype(jnp.bfloat16)
    g = jnp.dot(xb, wsg_ref[...].astype(jnp.bfloat16),
                preferred_element_type=jnp.float32)
    u = jnp.dot(xb, wsu_ref[...].astype(jnp.bfloat16),
                preferred_element_type=jnp.float32)
    h = (g * lax.logistic(g) * u).astype(jnp.bfloat16)
    y_ref[...] = jnp.dot(h, wsd_ref[...].astype(jnp.bfloat16),
                         preferred_element_type=jnp.float32)


def _shared_call(x2, wsg, wsu, wsd):
    return pl.pallas_call(
        _shared_kernel,
        grid=(S // _TB,),
        in_specs=[
            pl.BlockSpec((_TB, D), lambda g: (g, 0)),
            pl.BlockSpec((D, F), lambda g: (0, 0)),
            pl.BlockSpec((D, F), lambda g: (0, 0)),
            pl.BlockSpec((F, D), lambda g: (0, 0)),
        ],
        out_specs=pl.BlockSpec((_TB, D), lambda g: (g, 0)),
        out_shape=jax.ShapeDtypeStruct((S, D), jnp.float32),
    )(x2, wsg, wsu, wsd)


def _add_kernel(ysh_ref, yb_ref, y_ref):
    y_ref[...] = ysh_ref[...] + _unpack_f32(yb_ref[...])


def _add_call(ysh, yb):
    return pl.pallas_call(
        _add_kernel,
        grid=(S // _TB,),
        in_specs=[
            pl.BlockSpec((_TB, D), lambda g: (g, 0)),
            pl.BlockSpec((_TB, D // 2), lambda g: (g, 0)),
        ],
        out_specs=pl.BlockSpec((_TB, D), lambda g: (g, 0)),
        out_shape=jax.ShapeDtypeStruct((S, D), jnp.float32),
    )(ysh, yb)


def kernel(x, Wr, Wg, Wu, Wd, Wsg, Wsu, Wsd):
    x2 = x.reshape(S, D)
    _sc_gather, _sc_scatter = _sc_kernels()
    pos, dst, bexp, xp = _router_call(x2, Wr)
    ysh = _shared_call(x2, Wsg, Wsu, Wsd)
    xs = _sc_gather(pos.reshape(NW, NCG, CG), xp)
    ys = _gmm_call(bexp, xs, Wg, Wu, Wd)
    ybuf = _sc_scatter(ys, dst.reshape(NW, NCH, CH))
    y = _add_call(ysh, ybuf[:S])
    return y.reshape(1, S, D)
